# Initial kernel scaffold; baseline (speedup 1.0000x reference)
import jax
import jax.numpy as jnp
from jax.experimental import pallas as pl


def kernel(user_emb, item_emb, adj_vals, adj_rows, adj_cols):
    raise NotImplementedError("write your pallas kernel here")


# speedup vs baseline: 2.4125x; 2.4125x over previous
import functools

import jax
import jax.numpy as jnp
from jax import lax
from jax.experimental import pallas as pl
from jax.experimental.pallas import tpu as pltpu
from jax.experimental.pallas import tpu_sc as plsc

N_USERS = 10000
N_ITEMS = 40000
N = N_USERS + N_ITEMS
E = 800000
D = 64

NC = 2
NS = 16
H = N // NC
TRASH = H
ACC_ROWS = H + 88
E_TILE = E // NS
B = 128
S = 2000
NR = E_TILE // S
NB_FULL = 15
TAIL = S - NB_FULL * B

Z_PER_TILE = ACC_ROWS // NS
CP_PER_TILE = 1560


def _bcast_lane(v16, e):
    idx = jnp.full((16, 1), e, jnp.int32)
    return lax.gather(
        v16, idx,
        dimension_numbers=lax.GatherDimensionNumbers(
            offset_dims=(), collapsed_slice_dims=(0,), start_index_map=(0,)),
        slice_sizes=(1,),
        mode=lax.GatherScatterMode.PROMISE_IN_BOUNDS)


def _layer_body(x, vals, rows, cols, y,
                colc, dstc, valc, rowsb, lidxb,
                rowst, lidxt, zbuf, acc, sem):
    c = lax.axis_index("c")
    s = lax.axis_index("s")
    base_dst = c * H
    tile_e0 = s * E_TILE

    def zrow(r, _):
        for k in range(4):
            zbuf[r, pl.ds(k * 16, 16)] = jnp.zeros((16,), jnp.float32)
        return 0
    lax.fori_loop(0, 16, zrow, 0)
    z0 = s * Z_PER_TILE
    def zcopy(i, _):
        pltpu.sync_copy(zbuf, acc.at[pl.ds(z0 + i * 16, 16)])
        return 0
    lax.fori_loop(0, Z_PER_TILE // 16, zcopy, 0)
    plsc.subcore_barrier()

    def do_batch(off, nb, colref, dstref, valref, rowsref, lidxref):
        nedge = nb * 16
        pltpu.async_copy(
            x.at[colref.at[pl.ds(off, nedge)]], rowsref, sem).wait()
        def group(g, _):
            gb = off + g * 16
            d16 = dstref[pl.ds(gb, 16)]
            inr = (d16 >= base_dst) & (d16 < base_dst + H)
            lidxref[pl.ds(g * 16, 16)] = jnp.where(inr, d16 - base_dst, TRASH)
            v16 = valref[pl.ds(gb, 16)]
            for e in range(16):
                sv = _bcast_lane(v16, e)
                r = g * 16 + e
                for k in range(4):
                    rowsref[r, pl.ds(k * 16, 16)] = (
                        rowsref[r, pl.ds(k * 16, 16)] * sv)
            return 0
        lax.fori_loop(0, nb, group, 0)
        pltpu.sync_copy(rowsref, acc.at[lidxref], add=True)

    def sround(r, _):
        e0 = tile_e0 + r * S
        pltpu.sync_copy(cols.at[pl.ds(e0, S)], colc)
        pltpu.sync_copy(rows.at[pl.ds(e0, S)], dstc)
        pltpu.sync_copy(vals.at[pl.ds(e0, S)], valc)
        def batch(i, _):
            do_batch(i * B, B // 16, colc, dstc, valc, rowsb, lidxb)
            return 0
        lax.fori_loop(0, NB_FULL, batch, 0)
        do_batch(NB_FULL * B, TAIL // 16, colc, dstc, valc, rowst, lidxt)
        return 0
    lax.fori_loop(0, NR, sround, 0)

    plsc.subcore_barrier()
    r0 = s * CP_PER_TILE
    pltpu.sync_copy(acc.at[pl.ds(r0, CP_PER_TILE)],
                    y.at[pl.ds(base_dst + r0, CP_PER_TILE)])
    @pl.when(s == 0)
    def _():
        pltpu.sync_copy(acc.at[pl.ds(NS * CP_PER_TILE, 40)],
                        y.at[pl.ds(base_dst + NS * CP_PER_TILE, 40)])


@functools.partial(jax.jit)
def _sc_layer(x, vals, rows, cols):
    mesh = plsc.VectorSubcoreMesh(
        core_axis_name="c", subcore_axis_name="s",
        num_cores=NC, num_subcores=NS)
    return pl.kernel(
        _layer_body,
        out_type=jax.ShapeDtypeStruct((N, D), jnp.float32),
        mesh=mesh,
        compiler_params=pltpu.CompilerParams(use_tc_tiling_on_sc=False),
        scratch_types=[
            pltpu.VMEM((S,), jnp.int32),
            pltpu.VMEM((S,), jnp.int32),
            pltpu.VMEM((S,), jnp.float32),
            pltpu.VMEM((B, D), jnp.float32),
            pltpu.VMEM((B,), jnp.int32),
            pltpu.VMEM((TAIL, D), jnp.float32),
            pltpu.VMEM((TAIL,), jnp.int32),
            pltpu.VMEM((16, D), jnp.float32),
            pltpu.VMEM_SHARED((ACC_ROWS, D), jnp.float32),
            pltpu.SemaphoreType.DMA,
        ],
    )(x, vals, rows, cols)


def _mean_body(a, b, c, d, o):
    o[...] = (a[...] + b[...] + c[...] + d[...]) * 0.25


def _mean4(x0, x1, x2, x3):
    xs = [v.reshape(N // 2, 2 * D) for v in (x0, x1, x2, x3)]
    spec = pl.BlockSpec((5000, 2 * D), lambda i: (i, 0))
    out = pl.pallas_call(
        _mean_body,
        grid=(5,),
        in_specs=[spec] * 4,
        out_specs=spec,
        out_shape=jax.ShapeDtypeStruct((N // 2, 2 * D), jnp.float32),
    )(*xs)
    return out.reshape(N, D)


def kernel(user_emb, item_emb, adj_vals, adj_rows, adj_cols):
    x0 = jnp.concatenate([user_emb, item_emb], axis=0)
    x1 = _sc_layer(x0, adj_vals, adj_rows, adj_cols)
    x2 = _sc_layer(x1, adj_vals, adj_rows, adj_cols)
    x3 = _sc_layer(x2, adj_vals, adj_rows, adj_cols)
    out = _mean4(x0, x1, x2, x3)
    return (out[:N_USERS], out[N_USERS:])


# speedup vs baseline: 2.7581x; 1.1433x over previous
import jax
import jax.numpy as jnp
from jax import lax
from jax.experimental import pallas as pl
from jax.experimental.pallas import tpu as pltpu
from jax.experimental.pallas import tpu_sc as plsc

N_USERS = 10000
N_ITEMS = 40000
N = N_USERS + N_ITEMS
E = 800000
D = 64

NC = 2
NS = 16
H = N // NC
TRASH = H
ACC_ROWS = H + 88
B = 128
NB = 390
EXTRA_BASE = NS * NB * B

Z_PER_TILE = ACC_ROWS // NS
CP_PER_TILE = 1560


def _bcast_lane(v16, e):
    idx = jnp.full((16, 1), e, jnp.int32)
    return lax.gather(
        v16, idx,
        dimension_numbers=lax.GatherDimensionNumbers(
            offset_dims=(), collapsed_slice_dims=(0,), start_index_map=(0,)),
        slice_sizes=(1,),
        mode=lax.GatherScatterMode.PROMISE_IN_BOUNDS)


def _layer_body(x, vals, rows, cols, y,
                colb0, colb1, dstb0, dstb1, valb0, valb1,
                rowsb0, rowsb1, lidxb0, lidxb1, zbuf, acc,
                stg0, stg1, gat0, gat1, sct0, sct1):
    c = lax.axis_index("c")
    s = lax.axis_index("s")
    base_dst = c * H
    tile_base = s * (NB * B)

    colb = (colb0, colb1)
    dstb = (dstb0, dstb1)
    valb = (valb0, valb1)
    rowsb = (rowsb0, rowsb1)
    lidxb = (lidxb0, lidxb1)
    stg = (stg0, stg1)
    gat = (gat0, gat1)
    sct = (sct0, sct1)

    def zrow(r, _):
        for k in range(4):
            zbuf[r, pl.ds(k * 16, 16)] = jnp.zeros((16,), jnp.float32)
        return 0
    lax.fori_loop(0, 16, zrow, 0)
    z0 = s * Z_PER_TILE
    def zcopy(i, _):
        pltpu.sync_copy(zbuf, acc.at[pl.ds(z0 + i * 16, 16)])
        return 0
    lax.fori_loop(0, Z_PER_TILE // 16, zcopy, 0)
    plsc.subcore_barrier()

    def stage(bidx, p):
        e0 = tile_base + bidx * B
        pltpu.async_copy(cols.at[pl.ds(e0, B)], colb[p], stg[p])
        pltpu.async_copy(rows.at[pl.ds(e0, B)], dstb[p], stg[p])
        pltpu.async_copy(vals.at[pl.ds(e0, B)], valb[p], stg[p])

    def wait_stage(p):
        pltpu.make_async_copy(cols.at[pl.ds(0, B)], colb[p], stg[p]).wait()
        pltpu.make_async_copy(rows.at[pl.ds(0, B)], dstb[p], stg[p]).wait()
        pltpu.make_async_copy(vals.at[pl.ds(0, B)], valb[p], stg[p]).wait()

    def gather(p):
        pltpu.async_copy(x.at[colb[p]], rowsb[p], gat[p])

    def wait_gather(p):
        pltpu.make_async_copy(x.at[colb[p]], rowsb[p], gat[p]).wait()

    def scatter(p):
        pltpu.async_copy(rowsb[p], acc.at[lidxb[p]], sct[p], add=True)

    def wait_scatter(p):
        pltpu.make_async_copy(rowsb[p], acc.at[lidxb[p]], sct[p]).wait()

    def compute(p):
        def group(g, _):
            gb = g * 16
            d16 = dstb[p][pl.ds(gb, 16)]
            inr = (d16 >= base_dst) & (d16 < base_dst + H)
            lidxb[p][pl.ds(gb, 16)] = jnp.where(inr, d16 - base_dst, TRASH)
            v16 = valb[p][pl.ds(gb, 16)]
            for e in range(16):
                sv = _bcast_lane(v16, e)
                r = gb + e
                for k in range(4):
                    rowsb[p][r, pl.ds(k * 16, 16)] = (
                        rowsb[p][r, pl.ds(k * 16, 16)] * sv)
            return 0
        lax.fori_loop(0, B // 16, group, 0)

    stage(0, 0)
    stage(1, 1)
    wait_stage(0)
    gather(0)

    def pair(i, _):
        wait_gather(0)
        compute(0)
        scatter(0)
        stage(2 * i + 2, 0)
        @pl.when(i > 0)
        def _():
            wait_scatter(1)
        wait_stage(1)
        gather(1)
        wait_gather(1)
        compute(1)
        scatter(1)
        stage(2 * i + 3, 1)
        wait_scatter(0)
        wait_stage(0)
        @pl.when(i < NB // 2 - 1)
        def _():
            gather(0)
        return 0
    lax.fori_loop(0, NB // 2, pair, 0)

    wait_scatter(1)
    wait_stage(1)

    @pl.when(s < (E - EXTRA_BASE) // B)
    def _():
        e0 = EXTRA_BASE + s * B
        pltpu.sync_copy(cols.at[pl.ds(e0, B)], colb[0])
        pltpu.sync_copy(rows.at[pl.ds(e0, B)], dstb[0])
        pltpu.sync_copy(vals.at[pl.ds(e0, B)], valb[0])
        pltpu.async_copy(x.at[colb[0]], rowsb[0], gat[0]).wait()
        compute(0)
        pltpu.sync_copy(rowsb[0], acc.at[lidxb[0]], add=True)

    plsc.subcore_barrier()
    r0 = s * CP_PER_TILE
    pltpu.sync_copy(acc.at[pl.ds(r0, CP_PER_TILE)],
                    y.at[pl.ds(base_dst + r0, CP_PER_TILE)])
    @pl.when(s == 0)
    def _():
        pltpu.sync_copy(acc.at[pl.ds(NS * CP_PER_TILE, 40)],
                        y.at[pl.ds(base_dst + NS * CP_PER_TILE, 40)])


def _sc_layer(x, vals, rows, cols):
    mesh = plsc.VectorSubcoreMesh(
        core_axis_name="c", subcore_axis_name="s",
        num_cores=NC, num_subcores=NS)
    return pl.kernel(
        _layer_body,
        out_type=jax.ShapeDtypeStruct((N, D), jnp.float32),
        mesh=mesh,
        compiler_params=pltpu.CompilerParams(use_tc_tiling_on_sc=False),
        scratch_types=[
            pltpu.VMEM((B,), jnp.int32),
            pltpu.VMEM((B,), jnp.int32),
            pltpu.VMEM((B,), jnp.int32),
            pltpu.VMEM((B,), jnp.int32),
            pltpu.VMEM((B,), jnp.float32),
            pltpu.VMEM((B,), jnp.float32),
            pltpu.VMEM((B, D), jnp.float32),
            pltpu.VMEM((B, D), jnp.float32),
            pltpu.VMEM((B,), jnp.int32),
            pltpu.VMEM((B,), jnp.int32),
            pltpu.VMEM((16, D), jnp.float32),
            pltpu.VMEM_SHARED((ACC_ROWS, D), jnp.float32),
            pltpu.SemaphoreType.DMA,
            pltpu.SemaphoreType.DMA,
            pltpu.SemaphoreType.DMA,
            pltpu.SemaphoreType.DMA,
            pltpu.SemaphoreType.DMA,
            pltpu.SemaphoreType.DMA,
        ],
    )(x, vals, rows, cols)


def _mean_body(a, b, c, d, o):
    o[...] = (a[...] + b[...] + c[...] + d[...]) * 0.25


def _mean4(x0, x1, x2, x3):
    xs = [v.reshape(N // 2, 2 * D) for v in (x0, x1, x2, x3)]
    spec = pl.BlockSpec((5000, 2 * D), lambda i: (i, 0))
    out = pl.pallas_call(
        _mean_body,
        grid=(5,),
        in_specs=[spec] * 4,
        out_specs=spec,
        out_shape=jax.ShapeDtypeStruct((N // 2, 2 * D), jnp.float32),
    )(*xs)
    return out.reshape(N, D)


def kernel(user_emb, item_emb, adj_vals, adj_rows, adj_cols):
    x0 = jnp.concatenate([user_emb, item_emb], axis=0)
    x1 = _sc_layer(x0, adj_vals, adj_rows, adj_cols)
    x2 = _sc_layer(x1, adj_vals, adj_rows, adj_cols)
    x3 = _sc_layer(x2, adj_vals, adj_rows, adj_cols)
    out = _mean4(x0, x1, x2, x3)
    return (out[:N_USERS], out[N_USERS:])


# speedup vs baseline: 5.8887x; 2.1350x over previous
import jax
import jax.numpy as jnp
from jax import lax
from jax.experimental import pallas as pl
from jax.experimental.pallas import tpu as pltpu
from jax.experimental.pallas import tpu_sc as plsc

N_USERS = 10000
N_ITEMS = 40000
N = N_USERS + N_ITEMS
E = 800000
D = 64

NC = 2
NS = 16
H = N // NC
TRASH = H
ACC_ROWS = H + 88
B = 128
NB = 390
EXTRA_BASE = NS * NB * B

Z_PER_TILE = ACC_ROWS // NS
CP_PER_TILE = 1560


def _bcast_lane(v16, e):
    idx = jnp.full((16, 1), e, jnp.int32)
    return lax.gather(
        v16, idx,
        dimension_numbers=lax.GatherDimensionNumbers(
            offset_dims=(), collapsed_slice_dims=(0,), start_index_map=(0,)),
        slice_sizes=(1,),
        mode=lax.GatherScatterMode.PROMISE_IN_BOUNDS)


def _layer_body(x, vals, rows, cols, y,
                colb0, colb1, dstb0, dstb1, valb0, valb1,
                rowsb0, rowsb1, lidxb0, lidxb1, zbuf, acc,
                stg0, stg1, gat0, gat1, sct0, sct1):
    c = lax.axis_index("c")
    s = lax.axis_index("s")
    base_dst = c * H
    tile_base = s * (NB * B)

    colb = (colb0, colb1)
    dstb = (dstb0, dstb1)
    valb = (valb0, valb1)
    rowsb = (rowsb0, rowsb1)
    lidxb = (lidxb0, lidxb1)
    stg = (stg0, stg1)
    gat = (gat0, gat1)
    sct = (sct0, sct1)

    def zrow(r, _):
        for k in range(4):
            zbuf[r, pl.ds(k * 16, 16)] = jnp.zeros((16,), jnp.float32)
        return 0
    lax.fori_loop(0, 16, zrow, 0)
    z0 = s * Z_PER_TILE
    def zcopy(i, _):
        pltpu.sync_copy(zbuf, acc.at[pl.ds(z0 + i * 16, 16)])
        return 0
    lax.fori_loop(0, Z_PER_TILE // 16, zcopy, 0)
    plsc.subcore_barrier()

    def stage(bidx, p):
        e0 = tile_base + bidx * B
        pltpu.async_copy(cols.at[pl.ds(e0, B)], colb[p], stg[p])
        pltpu.async_copy(rows.at[pl.ds(e0, B)], dstb[p], stg[p])
        pltpu.async_copy(vals.at[pl.ds(e0, B)], valb[p], stg[p])

    def wait_stage(p):
        pltpu.make_async_copy(cols.at[pl.ds(0, B)], colb[p], stg[p]).wait()
        pltpu.make_async_copy(rows.at[pl.ds(0, B)], dstb[p], stg[p]).wait()
        pltpu.make_async_copy(vals.at[pl.ds(0, B)], valb[p], stg[p]).wait()

    def gather(p):
        pltpu.async_copy(x.at[colb[p]], rowsb[p], gat[p])

    def wait_gather(p):
        pltpu.make_async_copy(x.at[colb[p]], rowsb[p], gat[p]).wait()

    def scatter(p):
        pltpu.async_copy(rowsb[p], acc.at[lidxb[p]], sct[p], add=True)

    def wait_scatter(p):
        pltpu.make_async_copy(rowsb[p], acc.at[lidxb[p]], sct[p]).wait()

    def compute(p):
        for g in range(B // 16):
            gb = g * 16
            d16 = dstb[p][pl.ds(gb, 16)]
            inr = (d16 >= base_dst) & (d16 < base_dst + H)
            lidxb[p][pl.ds(gb, 16)] = jnp.where(inr, d16 - base_dst, TRASH)
            v16 = valb[p][pl.ds(gb, 16)]
            for e in range(16):
                sv = _bcast_lane(v16, e)
                r = gb + e
                for k in range(4):
                    rowsb[p][r, pl.ds(k * 16, 16)] = (
                        rowsb[p][r, pl.ds(k * 16, 16)] * sv)

    stage(0, 0)
    stage(1, 1)
    wait_stage(0)
    gather(0)

    def pair(i, _):
        wait_gather(0)
        compute(0)
        scatter(0)
        stage(2 * i + 2, 0)
        @pl.when(i > 0)
        def _():
            wait_scatter(1)
        wait_stage(1)
        gather(1)
        wait_gather(1)
        compute(1)
        scatter(1)
        stage(2 * i + 3, 1)
        wait_scatter(0)
        wait_stage(0)
        @pl.when(i < NB // 2 - 1)
        def _():
            gather(0)
        return 0
    lax.fori_loop(0, NB // 2, pair, 0)

    wait_scatter(1)
    wait_stage(1)

    @pl.when(s < (E - EXTRA_BASE) // B)
    def _():
        e0 = EXTRA_BASE + s * B
        pltpu.sync_copy(cols.at[pl.ds(e0, B)], colb[0])
        pltpu.sync_copy(rows.at[pl.ds(e0, B)], dstb[0])
        pltpu.sync_copy(vals.at[pl.ds(e0, B)], valb[0])
        pltpu.async_copy(x.at[colb[0]], rowsb[0], gat[0]).wait()
        compute(0)
        pltpu.sync_copy(rowsb[0], acc.at[lidxb[0]], add=True)

    plsc.subcore_barrier()
    r0 = s * CP_PER_TILE
    pltpu.sync_copy(acc.at[pl.ds(r0, CP_PER_TILE)],
                    y.at[pl.ds(base_dst + r0, CP_PER_TILE)])
    @pl.when(s == 0)
    def _():
        pltpu.sync_copy(acc.at[pl.ds(NS * CP_PER_TILE, 40)],
                        y.at[pl.ds(base_dst + NS * CP_PER_TILE, 40)])


def _sc_layer(x, vals, rows, cols):
    mesh = plsc.VectorSubcoreMesh(
        core_axis_name="c", subcore_axis_name="s",
        num_cores=NC, num_subcores=NS)
    return pl.kernel(
        _layer_body,
        out_type=jax.ShapeDtypeStruct((N, D), jnp.float32),
        mesh=mesh,
        compiler_params=pltpu.CompilerParams(use_tc_tiling_on_sc=False),
        scratch_types=[
            pltpu.VMEM((B,), jnp.int32),
            pltpu.VMEM((B,), jnp.int32),
            pltpu.VMEM((B,), jnp.int32),
            pltpu.VMEM((B,), jnp.int32),
            pltpu.VMEM((B,), jnp.float32),
            pltpu.VMEM((B,), jnp.float32),
            pltpu.VMEM((B, D), jnp.float32),
            pltpu.VMEM((B, D), jnp.float32),
            pltpu.VMEM((B,), jnp.int32),
            pltpu.VMEM((B,), jnp.int32),
            pltpu.VMEM((16, D), jnp.float32),
            pltpu.VMEM_SHARED((ACC_ROWS, D), jnp.float32),
            pltpu.SemaphoreType.DMA,
            pltpu.SemaphoreType.DMA,
            pltpu.SemaphoreType.DMA,
            pltpu.SemaphoreType.DMA,
            pltpu.SemaphoreType.DMA,
            pltpu.SemaphoreType.DMA,
        ],
    )(x, vals, rows, cols)


def _mean_body(a, b, c, d, o):
    o[...] = (a[...] + b[...] + c[...] + d[...]) * 0.25


def _mean4(x0, x1, x2, x3):
    xs = [v.reshape(N // 2, 2 * D) for v in (x0, x1, x2, x3)]
    spec = pl.BlockSpec((5000, 2 * D), lambda i: (i, 0))
    out = pl.pallas_call(
        _mean_body,
        grid=(5,),
        in_specs=[spec] * 4,
        out_specs=spec,
        out_shape=jax.ShapeDtypeStruct((N // 2, 2 * D), jnp.float32),
    )(*xs)
    return out.reshape(N, D)


def kernel(user_emb, item_emb, adj_vals, adj_rows, adj_cols):
    x0 = jnp.concatenate([user_emb, item_emb], axis=0)
    x1 = _sc_layer(x0, adj_vals, adj_rows, adj_cols)
    x2 = _sc_layer(x1, adj_vals, adj_rows, adj_cols)
    x3 = _sc_layer(x2, adj_vals, adj_rows, adj_cols)
    out = _mean4(x0, x1, x2, x3)
    return (out[:N_USERS], out[N_USERS:])
